# baseline (speedup 1.0000x reference)
import functools

import jax
import jax.numpy as jnp
from jax import lax
from jax.experimental import pallas as pl
from jax.experimental.pallas import tpu as pltpu
from jax.experimental.pallas import tpu_sc as plsc

VOCAB = 100000
EMBED_DIM = 64
CONTEXT = 20
BATCH = 4096
HIDDEN = 128
FAN1 = CONTEXT * EMBED_DIM
HID_AUG = HIDDEN + 1
VOCAB_PAD = 100352
N_PAD = VOCAB_PAD - VOCAB

NUM_WORKERS = 32
TOTAL_LOOKUPS = BATCH * CONTEXT
LOOKUPS_PER_WORKER = TOTAL_LOOKUPS // NUM_WORKERS
IDX_CHUNK = 128
CHUNKS_PER_WORKER = LOOKUPS_PER_WORKER // IDX_CHUNK
HALF = CHUNKS_PER_WORKER // 2

VCW = 2048
VCC = 1024
VC = 512


def _sc_gather_kernel(idx_hbm, table_hbm, out_hbm, idx_v, rows_v, sem):
    wid = lax.axis_index("s") * 2 + lax.axis_index("c")
    pltpu.sync_copy(idx_hbm.at[pl.ds(wid * LOOKUPS_PER_WORKER,
                                     LOOKUPS_PER_WORKER)], idx_v)
    for half in range(2):
        cps = []
        for j in range(HALF):
            chunk = half * HALF + j
            cps.append(
                pltpu.async_copy(
                    table_hbm.at[idx_v.at[pl.ds(chunk * IDX_CHUNK, IDX_CHUNK)]],
                    rows_v.at[pl.ds(j * IDX_CHUNK, IDX_CHUNK)],
                    sem,
                )
            )
        for cp in cps:
            cp.wait()
        base = wid * LOOKUPS_PER_WORKER + half * HALF * IDX_CHUNK
        pltpu.sync_copy(rows_v, out_hbm.at[pl.ds(base, HALF * IDX_CHUNK)])


def _sc_gather(idx, table):
    mesh = plsc.VectorSubcoreMesh(core_axis_name="c", subcore_axis_name="s")
    return pl.kernel(
        _sc_gather_kernel,
        mesh=mesh,
        out_type=jax.ShapeDtypeStruct((TOTAL_LOOKUPS, EMBED_DIM), jnp.float32),
        scratch_types=[
            pltpu.VMEM((LOOKUPS_PER_WORKER,), jnp.int32),
            pltpu.VMEM((HALF * IDX_CHUNK, EMBED_DIM), jnp.float32),
            pltpu.SemaphoreType.DMA,
        ],
        compiler_params=pltpu.CompilerParams(use_tc_tiling_on_sc=False),
    )(idx, table)


def _w2_kernel(w2_ref, b2_ref, out_ref):
    v = pl.program_id(0)
    wt = w2_ref[...].T.astype(jnp.bfloat16)
    col = lax.broadcasted_iota(jnp.int32, (HIDDEN, VCW), 1) + v * VCW
    out_ref[pl.ds(0, HIDDEN), :] = jnp.where(col < VOCAB, wt, 0)
    b2row = b2_ref[...].reshape(1, VCW).astype(jnp.bfloat16)
    col1 = lax.broadcasted_iota(jnp.int32, (1, VCW), 1) + v * VCW
    out_ref[pl.ds(HIDDEN, 1), :] = jnp.where(col1 < VOCAB, b2row, 0)


def _tc_w2taug(W2, b2):
    return pl.pallas_call(
        _w2_kernel,
        grid=(VOCAB_PAD // VCW,),
        in_specs=[
            pl.BlockSpec((VCW, HIDDEN), lambda i: (i, 0)),
            pl.BlockSpec((VCW,), lambda i: (i,)),
        ],
        out_specs=pl.BlockSpec((HID_AUG, VCW), lambda i: (0, i)),
        out_shape=jax.ShapeDtypeStruct((HID_AUG, VOCAB_PAD), jnp.bfloat16),
    )(W2, b2)


def _ht_kernel(emb_ref, w1_ref, b1_ref, out_ref):
    e = emb_ref[...].astype(jnp.bfloat16)
    ht = lax.dot_general(w1_ref[...], e, (((1,), (1,)), ((), ())),
                         preferred_element_type=jnp.float32)
    out_ref[pl.ds(0, HIDDEN), :] = jnp.maximum(ht + b1_ref[...], 0.0)
    out_ref[pl.ds(HIDDEN, 1), :] = jnp.ones((1, BATCH), jnp.float32)


def _tc_ht(embeds, W1b, b1c):
    return pl.pallas_call(
        _ht_kernel,
        grid=(1,),
        in_specs=[
            pl.BlockSpec((BATCH, FAN1), lambda i: (0, 0)),
            pl.BlockSpec((HIDDEN, FAN1), lambda i: (0, 0)),
            pl.BlockSpec((HIDDEN, 1), lambda i: (0, 0)),
        ],
        out_specs=pl.BlockSpec((HID_AUG, BATCH), lambda i: (0, 0)),
        out_shape=jax.ShapeDtypeStruct((HID_AUG, BATCH), jnp.float32),
    )(embeds, W1b, b1c)


HB = BATCH // 2
NV = VOCAB_PAD // VC


def _cb_kernel(w2t_ref, ht_ref, out_ref, s_ref, norm_ref):
    p = pl.program_id(0)
    v = pl.program_id(1)

    def chunk(h):
        ht = ht_ref[:, pl.ds(h * HB, HB)].astype(jnp.bfloat16)
        return lax.dot_general(w2t_ref[:, pl.ds(v * VC, VC)], ht,
                               (((0,), (0,)), ((), ())),
                               preferred_element_type=jnp.float32)

    def c_step(h):
        part = jnp.sum(jnp.exp(chunk(h)), axis=0, keepdims=True)

        @pl.when(v == 0)
        def _():
            s_ref[pl.ds(h, 1), :] = part

        @pl.when(v > 0)
        def _():
            s_ref[pl.ds(h, 1), :] += part

    def b_step(h):
        @pl.when(v == 0)
        def _():
            norm_ref[pl.ds(h, 1), :] = jnp.log(
                s_ref[pl.ds(h, 1), :] - float(N_PAD))

        out_ref[...] = chunk(h) - norm_ref[pl.ds(h, 1), :]

    @pl.when(p == 0)
    def _():
        c_step(0)

    @pl.when(p == 1)
    def _():
        b_step(0)
        c_step(1)

    @pl.when(p == 2)
    def _():
        b_step(1)


def _tc_cb(W2taug, hT):
    return pl.pallas_call(
        _cb_kernel,
        grid=(3, NV),
        in_specs=[
            pl.BlockSpec((HID_AUG, VOCAB_PAD), lambda p, v: (0, 0)),
            pl.BlockSpec((HID_AUG, BATCH), lambda p, v: (0, 0)),
        ],
        out_specs=pl.BlockSpec(
            (VC, HB),
            lambda p, v: (jnp.where(p == 0, 0, v), jnp.where(p == 2, 1, 0)),
        ),
        out_shape=jax.ShapeDtypeStruct((VOCAB, BATCH), jnp.float32),
        scratch_shapes=[
            pltpu.VMEM((2, HB), jnp.float32),
            pltpu.VMEM((2, HB), jnp.float32),
        ],
        compiler_params=pltpu.CompilerParams(
            dimension_semantics=("arbitrary", "arbitrary"),
        ),
    )(W2taug, hT)


def kernel(inputs, emb_table, W1, b1, W2, b2):
    idx = inputs.reshape(TOTAL_LOOKUPS)
    embeds = _sc_gather(idx, emb_table).reshape(BATCH, FAN1)
    W1b = W1.astype(jnp.bfloat16)
    W2taug = _tc_w2taug(W2, b2)
    hT = _tc_ht(embeds, W1b, b1.reshape(HIDDEN, 1))
    out_t = _tc_cb(W2taug, hT)
    return out_t.T


# speedup vs baseline: 1.1232x; 1.1232x over previous
import functools

import jax
import jax.numpy as jnp
from jax import lax
from jax.experimental import pallas as pl
from jax.experimental.pallas import tpu as pltpu
from jax.experimental.pallas import tpu_sc as plsc

VOCAB = 100000
EMBED_DIM = 64
CONTEXT = 20
BATCH = 4096
HIDDEN = 128
FAN1 = CONTEXT * EMBED_DIM
HID_AUG = HIDDEN + 1
VOCAB_PAD = 100352
N_PAD = VOCAB_PAD - VOCAB

NUM_WORKERS = 32
TOTAL_LOOKUPS = BATCH * CONTEXT
LOOKUPS_PER_WORKER = TOTAL_LOOKUPS // NUM_WORKERS
IDX_CHUNK = 128
CHUNKS_PER_WORKER = LOOKUPS_PER_WORKER // IDX_CHUNK
HALF = CHUNKS_PER_WORKER // 2

VCW = 2048
VCC = 1024
VC = 1024


def _sc_gather_kernel(idx_hbm, table_hbm, out_hbm, idx_v, rows_v, sem):
    wid = lax.axis_index("s") * 2 + lax.axis_index("c")
    pltpu.sync_copy(idx_hbm.at[pl.ds(wid * LOOKUPS_PER_WORKER,
                                     LOOKUPS_PER_WORKER)], idx_v)
    for half in range(2):
        cps = []
        for j in range(HALF):
            chunk = half * HALF + j
            cps.append(
                pltpu.async_copy(
                    table_hbm.at[idx_v.at[pl.ds(chunk * IDX_CHUNK, IDX_CHUNK)]],
                    rows_v.at[pl.ds(j * IDX_CHUNK, IDX_CHUNK)],
                    sem,
                )
            )
        for cp in cps:
            cp.wait()
        base = wid * LOOKUPS_PER_WORKER + half * HALF * IDX_CHUNK
        pltpu.sync_copy(rows_v, out_hbm.at[pl.ds(base, HALF * IDX_CHUNK)])


def _sc_gather(idx, table):
    mesh = plsc.VectorSubcoreMesh(core_axis_name="c", subcore_axis_name="s")
    return pl.kernel(
        _sc_gather_kernel,
        mesh=mesh,
        out_type=jax.ShapeDtypeStruct((TOTAL_LOOKUPS, EMBED_DIM), jnp.float32),
        scratch_types=[
            pltpu.VMEM((LOOKUPS_PER_WORKER,), jnp.int32),
            pltpu.VMEM((HALF * IDX_CHUNK, EMBED_DIM), jnp.float32),
            pltpu.SemaphoreType.DMA,
        ],
        compiler_params=pltpu.CompilerParams(use_tc_tiling_on_sc=False),
    )(idx, table)


def _w2_kernel(w2_ref, b2_ref, out_ref):
    v = pl.program_id(0)
    wt = w2_ref[...].T.astype(jnp.bfloat16)
    col = lax.broadcasted_iota(jnp.int32, (HIDDEN, VCW), 1) + v * VCW
    out_ref[pl.ds(0, HIDDEN), :] = jnp.where(col < VOCAB, wt, 0)
    b2row = b2_ref[...].reshape(1, VCW).astype(jnp.bfloat16)
    col1 = lax.broadcasted_iota(jnp.int32, (1, VCW), 1) + v * VCW
    out_ref[pl.ds(HIDDEN, 1), :] = jnp.where(col1 < VOCAB, b2row, 0)


def _tc_w2taug(W2, b2):
    return pl.pallas_call(
        _w2_kernel,
        grid=(VOCAB_PAD // VCW,),
        in_specs=[
            pl.BlockSpec((VCW, HIDDEN), lambda i: (i, 0)),
            pl.BlockSpec((VCW,), lambda i: (i,)),
        ],
        out_specs=pl.BlockSpec((HID_AUG, VCW), lambda i: (0, i)),
        out_shape=jax.ShapeDtypeStruct((HID_AUG, VOCAB_PAD), jnp.bfloat16),
    )(W2, b2)


def _ht_kernel(emb_ref, w1_ref, b1_ref, out_ref):
    e = emb_ref[...].astype(jnp.bfloat16)
    ht = lax.dot_general(w1_ref[...], e, (((1,), (1,)), ((), ())),
                         preferred_element_type=jnp.float32)
    out_ref[pl.ds(0, HIDDEN), :] = jnp.maximum(ht + b1_ref[...], 0.0)
    out_ref[pl.ds(HIDDEN, 1), :] = jnp.ones((1, BATCH), jnp.float32)


def _tc_ht(embeds, W1b, b1c):
    return pl.pallas_call(
        _ht_kernel,
        grid=(1,),
        in_specs=[
            pl.BlockSpec((BATCH, FAN1), lambda i: (0, 0)),
            pl.BlockSpec((HIDDEN, FAN1), lambda i: (0, 0)),
            pl.BlockSpec((HIDDEN, 1), lambda i: (0, 0)),
        ],
        out_specs=pl.BlockSpec((HID_AUG, BATCH), lambda i: (0, 0)),
        out_shape=jax.ShapeDtypeStruct((HID_AUG, BATCH), jnp.float32),
    )(embeds, W1b, b1c)


HB = BATCH // 2
NV = VOCAB_PAD // VC


def _cb_kernel(w2t_ref, ht_ref, out_ref, s_ref, norm_ref):
    p = pl.program_id(0)
    v = pl.program_id(1)

    def chunk(h):
        ht = ht_ref[:, pl.ds(h * HB, HB)].astype(jnp.bfloat16)
        return lax.dot_general(w2t_ref[:, pl.ds(v * VC, VC)], ht,
                               (((0,), (0,)), ((), ())),
                               preferred_element_type=jnp.float32)

    def c_step(h):
        part = jnp.sum(jnp.exp(chunk(h)), axis=0, keepdims=True)

        @pl.when(v == 0)
        def _():
            s_ref[pl.ds(h, 1), :] = part

        @pl.when(v > 0)
        def _():
            s_ref[pl.ds(h, 1), :] += part

    def b_step(h):
        @pl.when(v == 0)
        def _():
            norm_ref[pl.ds(h, 1), :] = jnp.log(
                s_ref[pl.ds(h, 1), :] - float(N_PAD))

        out_ref[...] = chunk(h) - norm_ref[pl.ds(h, 1), :]

    @pl.when(p == 0)
    def _():
        c_step(0)

    @pl.when(p == 1)
    def _():
        b_step(0)
        c_step(1)

    @pl.when(p == 2)
    def _():
        b_step(1)


def _tc_cb(W2taug, hT):
    return pl.pallas_call(
        _cb_kernel,
        grid=(3, NV),
        in_specs=[
            pl.BlockSpec((HID_AUG, VOCAB_PAD), lambda p, v: (0, 0)),
            pl.BlockSpec((HID_AUG, BATCH), lambda p, v: (0, 0)),
        ],
        out_specs=pl.BlockSpec(
            (VC, HB),
            lambda p, v: (jnp.where(p == 0, 0, v), jnp.where(p == 2, 1, 0)),
        ),
        out_shape=jax.ShapeDtypeStruct((VOCAB, BATCH), jnp.float32),
        scratch_shapes=[
            pltpu.VMEM((2, HB), jnp.float32),
            pltpu.VMEM((2, HB), jnp.float32),
        ],
        compiler_params=pltpu.CompilerParams(
            dimension_semantics=("arbitrary", "arbitrary"),
        ),
    )(W2taug, hT)


def kernel(inputs, emb_table, W1, b1, W2, b2):
    idx = inputs.reshape(TOTAL_LOOKUPS)
    embeds = _sc_gather(idx, emb_table).reshape(BATCH, FAN1)
    W1b = W1.astype(jnp.bfloat16)
    W2taug = _tc_w2taug(W2, b2)
    hT = _tc_ht(embeds, W1b, b1.reshape(HIDDEN, 1))
    out_t = _tc_cb(W2taug, hT)
    return out_t.T
